# baseline (speedup 1.0000x reference)
import functools

import jax
import jax.numpy as jnp
from jax import lax
from jax.experimental import pallas as pl
from jax.experimental.pallas import tpu as pltpu
from jax.experimental.pallas import tpu_sc as plsc

_S = 10
_D = 32
_NF = 26
_FP = 32
_F1, _F2 = 20, 6
_B = 16384
_V = 100000
_PACK = 4
_MMROWS = _V // _PACK
_MMBLK = 1000
_NW = 32
_BPW = _B // _NW
_CHUNK = 128
_NJ = _NF * _BPW // _CHUNK
_FIRE = 8
_NG = _NJ // _FIRE
_QROWS = _BPW // _CHUNK
_NT = _CHUNK // 16
_TBLK = _B // 8


def _pmat_body(e_ref, w_ref, obs_ref, wo_ref, o_ref, ob_ref):
    e = e_ref[...].astype(jnp.bfloat16)
    o_ref[...] = jax.lax.dot_general(
        e, w_ref[...], (((1,), (0,)), ((), ())),
        preferred_element_type=jnp.float32)

    @pl.when(pl.program_id(0) == 0)
    def _():
        ob = obs_ref[...].astype(jnp.bfloat16).astype(jnp.float32)
        wo = wo_ref[...].astype(jnp.bfloat16).astype(jnp.float32)
        ob_ref[...] = jnp.sum(ob * wo, axis=1)


def _tidx_body(c1_ref, c2_ref, o_ref):
    f1 = jnp.swapaxes(c1_ref[...], 0, 1) * _FP \
        + lax.broadcasted_iota(jnp.int32, (_F1, _TBLK), 0)
    f2 = jnp.swapaxes(c2_ref[...], 0, 1) * _FP \
        + lax.broadcasted_iota(jnp.int32, (_F2, _TBLK), 0) + _F1
    o_ref[0:_F1, :] = f1
    o_ref[_F1:_NF, :] = f2


def _sc_body(pflat, fidx, base, out, idx_v, vals_v, base_v, out_v, sem):
    wid = lax.axis_index("s") * 2 + lax.axis_index("c")
    row0 = wid * _BPW
    pltpu.sync_copy(fidx.at[:, pl.ds(row0, _BPW)], idx_v)
    pltpu.sync_copy(base.at[pl.ds(row0, _BPW)], base_v)

    def _idx_slice(j):
        return idx_v.at[j // _QROWS, pl.ds((j % _QROWS) * _CHUNK, _CHUNK)]

    for k in range(_FIRE):
        pltpu.async_copy(pflat.at[_idx_slice(k)], vals_v.at[k], sem)

    def fire_drain(g, carry):
        j0 = (g + 1) * _FIRE
        for k in range(_FIRE):
            pltpu.async_copy(pflat.at[_idx_slice(j0 + k)], vals_v.at[j0 + k], sem)
        j1 = g * _FIRE
        for k in range(_FIRE):
            pltpu.make_async_copy(
                pflat.at[_idx_slice(j1 + k)], vals_v.at[j1 + k], sem).wait()
        return carry

    lax.fori_loop(0, _NG - 1, fire_drain, 0)
    j1 = (_NG - 1) * _FIRE
    for k in range(_FIRE):
        pltpu.make_async_copy(
            pflat.at[_idx_slice(j1 + k)], vals_v.at[j1 + k], sem).wait()

    for q in range(_QROWS):
        for t in range(_NT):
            lane = pl.ds(t * 16, 16)
            acc = vals_v[q, lane]
            for f in range(1, _NF):
                acc = acc + vals_v[f * _QROWS + q, lane]
            off = pl.ds(q * _CHUNK + t * 16, 16)
            out_v[off] = acc + base_v[off]
    pltpu.sync_copy(out_v, out.at[pl.ds(row0, _BPW)])


@jax.jit
def kernel(cate, cat_incre, obs, emb_table, fc_w, fc_b):
    w_feat = fc_w[0, : _NF * _D].reshape(_NF, _D).astype(jnp.bfloat16)
    w_pad = jnp.pad(w_feat, ((0, _FP - _NF), (0, 0)))
    w4 = jnp.kron(jnp.eye(_PACK, dtype=jnp.bfloat16), w_pad.T)

    emb4 = emb_table.reshape(_MMROWS, _PACK * _D)
    p4, obs_term = pl.pallas_call(
        _pmat_body,
        grid=(_MMROWS // _MMBLK,),
        in_specs=[
            pl.BlockSpec((_MMBLK, _PACK * _D), lambda i: (i, 0)),
            pl.BlockSpec((_PACK * _D, _PACK * _FP), lambda i: (0, 0)),
            pl.BlockSpec((_B, 13), lambda i: (0, 0)),
            pl.BlockSpec((1, 13), lambda i: (0, 0)),
        ],
        out_specs=[
            pl.BlockSpec((_MMBLK, _PACK * _FP), lambda i: (i, 0)),
            pl.BlockSpec((_B,), lambda i: (0,)),
        ],
        out_shape=[
            jax.ShapeDtypeStruct((_MMROWS, _PACK * _FP), jnp.float32),
            jax.ShapeDtypeStruct((_B,), jnp.float32),
        ],
    )(emb4, w4, obs, fc_w[:, _NF * _D:])
    pflat = p4.reshape(_V * _FP)

    fidx_t = pl.pallas_call(
        _tidx_body,
        grid=(8,),
        in_specs=[
            pl.BlockSpec((_TBLK, _F1), lambda i: (i, 0)),
            pl.BlockSpec((_TBLK, _F2), lambda i: (i, 0)),
        ],
        out_specs=pl.BlockSpec((_NF, _TBLK), lambda i: (0, i)),
        out_shape=jax.ShapeDtypeStruct((_NF, _B), jnp.int32),
    )(cate.astype(jnp.int32), cat_incre.astype(jnp.int32))

    sc = functools.partial(
        pl.kernel,
        mesh=plsc.VectorSubcoreMesh(core_axis_name="c", subcore_axis_name="s"),
        out_type=jax.ShapeDtypeStruct((_B,), jnp.float32),
        scratch_types=[
            pltpu.VMEM((_NF, _BPW), jnp.int32),
            pltpu.VMEM((_NJ, _CHUNK), jnp.float32),
            pltpu.VMEM((_BPW,), jnp.float32),
            pltpu.VMEM((_BPW,), jnp.float32),
            pltpu.SemaphoreType.DMA,
        ],
    )(_sc_body)
    logit = sc(pflat, fidx_t, obs_term)

    x = jnp.broadcast_to(logit[None, :, None], (_S, _B, 1)) + fc_b
    x = jax.lax.optimization_barrier(x)
    prob_ens = jax.nn.sigmoid(x).squeeze(-1)
    prob = prob_ens.mean(axis=0)
    prob_std = prob_ens.std(axis=0, ddof=1)
    return (prob, prob_std, emb_table, emb_table)


# speedup vs baseline: 1.1041x; 1.1041x over previous
import functools

import jax
import jax.numpy as jnp
from jax import lax
from jax.experimental import pallas as pl
from jax.experimental.pallas import tpu as pltpu
from jax.experimental.pallas import tpu_sc as plsc

_S = 10
_D = 32
_NF = 26
_FP = 32
_F1, _F2 = 20, 6
_B = 16384
_V = 100000
_PACK = 4
_MMROWS = _V // _PACK
_MMBLK = 1000
_NW = 32
_BPW = _B // _NW
_CHUNK = 128
_NJ = _NF * _BPW // _CHUNK
_FIRE = 8
_NG = _NJ // _FIRE
_QROWS = _BPW // _CHUNK
_NT = _CHUNK // 16


def _pmat_body(e_ref, w_ref, o_ref):
    e = e_ref[...].astype(jnp.bfloat16)
    o_ref[...] = jax.lax.dot_general(
        e, w_ref[...], (((1,), (0,)), ((), ())),
        preferred_element_type=jnp.float32)


def _obs_body(obs_ref, wo_ref, o_ref):
    ob = obs_ref[...].astype(jnp.bfloat16).astype(jnp.float32)
    wo = wo_ref[...].astype(jnp.bfloat16).astype(jnp.float32)
    o_ref[...] = jnp.sum(ob * wo, axis=1)


def _sc_body(pflat, fidx, base, out, idx_v, vals_v, base_v, out_v, sem):
    wid = lax.axis_index("s") * 2 + lax.axis_index("c")
    row0 = wid * _BPW
    pltpu.sync_copy(fidx.at[wid], idx_v)
    pltpu.sync_copy(base.at[pl.ds(row0, _BPW)], base_v)

    for k in range(_FIRE):
        pltpu.async_copy(pflat.at[idx_v.at[k]], vals_v.at[k], sem)

    def fire_drain(g, carry):
        j0 = (g + 1) * _FIRE
        for k in range(_FIRE):
            pltpu.async_copy(pflat.at[idx_v.at[j0 + k]], vals_v.at[j0 + k], sem)
        j1 = g * _FIRE
        for k in range(_FIRE):
            pltpu.make_async_copy(
                pflat.at[idx_v.at[j1 + k]], vals_v.at[j1 + k], sem).wait()
        return carry

    lax.fori_loop(0, _NG - 1, fire_drain, 0)
    j1 = (_NG - 1) * _FIRE
    for k in range(_FIRE):
        pltpu.make_async_copy(
            pflat.at[idx_v.at[j1 + k]], vals_v.at[j1 + k], sem).wait()

    for q in range(_QROWS):
        for t in range(_NT):
            lane = pl.ds(t * 16, 16)
            acc = vals_v[q, lane]
            for f in range(1, _NF):
                acc = acc + vals_v[f * _QROWS + q, lane]
            off = pl.ds(q * _CHUNK + t * 16, 16)
            out_v[off] = acc + base_v[off]
    pltpu.sync_copy(out_v, out.at[pl.ds(row0, _BPW)])


@jax.jit
def kernel(cate, cat_incre, obs, emb_table, fc_w, fc_b):
    w_feat = fc_w[0, : _NF * _D].reshape(_NF, _D).astype(jnp.bfloat16)
    w_pad = jnp.pad(w_feat, ((0, _FP - _NF), (0, 0)))
    w4 = jnp.kron(jnp.eye(_PACK, dtype=jnp.bfloat16), w_pad.T)

    idx = jnp.concatenate([cate, cat_incre], axis=1).astype(jnp.int32)
    fidx = idx * _FP + jnp.arange(_NF, dtype=jnp.int32)[None, :]
    fidx_w = (fidx.T.reshape(_NF, _NW, _BPW)
              .transpose(1, 0, 2)
              .reshape(_NW, _NJ, _CHUNK))

    emb4 = emb_table.reshape(_MMROWS, _PACK * _D)
    p4 = pl.pallas_call(
        _pmat_body,
        grid=(_MMROWS // _MMBLK,),
        in_specs=[
            pl.BlockSpec((_MMBLK, _PACK * _D), lambda i: (i, 0)),
            pl.BlockSpec((_PACK * _D, _PACK * _FP), lambda i: (0, 0)),
        ],
        out_specs=pl.BlockSpec((_MMBLK, _PACK * _FP), lambda i: (i, 0)),
        out_shape=jax.ShapeDtypeStruct((_MMROWS, _PACK * _FP), jnp.float32),
    )(emb4, w4)
    pflat = p4.reshape(_V * _FP)

    obs_term = pl.pallas_call(
        _obs_body,
        grid=(8,),
        in_specs=[
            pl.BlockSpec((_B // 8, 13), lambda i: (i, 0)),
            pl.BlockSpec((1, 13), lambda i: (0, 0)),
        ],
        out_specs=pl.BlockSpec((_B // 8,), lambda i: (i,)),
        out_shape=jax.ShapeDtypeStruct((_B,), jnp.float32),
    )(obs, fc_w[:, _NF * _D:])

    sc = functools.partial(
        pl.kernel,
        mesh=plsc.VectorSubcoreMesh(core_axis_name="c", subcore_axis_name="s"),
        out_type=jax.ShapeDtypeStruct((_B,), jnp.float32),
        scratch_types=[
            pltpu.VMEM((_NJ, _CHUNK), jnp.int32),
            pltpu.VMEM((_NJ, _CHUNK), jnp.float32),
            pltpu.VMEM((_BPW,), jnp.float32),
            pltpu.VMEM((_BPW,), jnp.float32),
            pltpu.SemaphoreType.DMA,
        ],
    )(_sc_body)
    logit = sc(pflat, fidx_w, obs_term)

    x = jnp.broadcast_to(logit[None, :, None], (_S, _B, 1)) + fc_b
    x = jax.lax.optimization_barrier(x)
    prob_ens = jax.nn.sigmoid(x).squeeze(-1)
    prob = prob_ens.mean(axis=0)
    prob_std = prob_ens.std(axis=0, ddof=1)
    return (prob, prob_std, emb_table, emb_table)
